# baseline (speedup 1.0000x reference)
import functools

import jax
import jax.numpy as jnp
from jax.experimental import pallas as pl
from jax.experimental.pallas import tpu as pltpu

_BLK = 1024


def _gate_block(x_ref, wt_ref, b_ref, o_ref):
    logits = jnp.dot(x_ref[...], wt_ref[...],
                     preferred_element_type=jnp.float32) + b_ref[...]
    l0 = logits[:, 0:1]
    lrest = jnp.max(logits[:, 1:], axis=1, keepdims=True)
    is0 = (l0 >= lrest).astype(jnp.float32)
    o_ref[...] = jnp.concatenate([is0, 1.0 - is0], axis=1)


@jax.jit
def kernel(x, W, b):
    B, S, D = x.shape
    K = W.shape[0]
    M = B * S
    x2 = x.reshape(M, D)
    wt = W.T
    b2 = b.reshape(1, K)
    out = pl.pallas_call(
        _gate_block,
        grid=(M // _BLK,),
        in_specs=[
            pl.BlockSpec((_BLK, D), lambda i: (i, 0)),
            pl.BlockSpec((D, K), lambda i: (0, 0)),
            pl.BlockSpec((1, K), lambda i: (0, 0)),
        ],
        out_specs=pl.BlockSpec((_BLK, 2), lambda i: (i, 0)),
        out_shape=jax.ShapeDtypeStruct((M, 2), jnp.float32),
        compiler_params=pltpu.CompilerParams(
            dimension_semantics=("arbitrary",),
        ),
    )(x2, wt, b2)
    return out.reshape(B, S, 2)
